# Initial kernel scaffold; baseline (speedup 1.0000x reference)
import functools

import jax
import jax.numpy as jnp
from jax import lax
from jax.experimental import pallas as pl
from jax.experimental.pallas import tpu as pltpu
from jax.experimental.pallas import tpu_sc as plsc

N_NODES = 10000
N_EDGES = 320000
D_FEAT = 128
N_CLASSES = 64

NC, NS = 2, 16
NW = NC * NS
E_W = N_EDGES // NW
CHUNK = 128
NFULL = E_W // CHUNK
REM = E_W - NFULL * CHUNK
RPT = N_NODES // NS
VPR = N_CLASSES // 16


def _mm_body(feat_ref, w_ref, o_ref):
    o_ref[...] = lax.dot_general(
        feat_ref[...], w_ref[...],
        (((1,), (1,)), ((), ())),
        preferred_element_type=jnp.float32,
    )


def _tc_matmul(feat, W):
    return pl.pallas_call(
        _mm_body,
        out_shape=jax.ShapeDtypeStruct((N_NODES, N_CLASSES), jnp.float32),
    )(feat, W)


def _comb_body(q_ref, b_ref, o_ref):
    o_ref[...] = q_ref[0] + q_ref[1] + b_ref[...]


def _tc_combine(q, b2):
    return pl.pallas_call(
        _comb_body,
        out_shape=jax.ShapeDtypeStruct((N_NODES, N_CLASSES), jnp.float32),
    )(q, b2)


def _make_sc_round(nprev: int):
    mesh = plsc.VectorSubcoreMesh(core_axis_name="c", subcore_axis_name="s")
    scratch = [
        pltpu.VMEM_SHARED((N_NODES, N_CLASSES), jnp.float32),
        pltpu.VMEM_SHARED((N_NODES, N_CLASSES), jnp.float32),
        pltpu.VMEM((RPT, N_CLASSES), jnp.float32),
        pltpu.VMEM((E_W,), jnp.int32),
        pltpu.VMEM((E_W,), jnp.int32),
        pltpu.VMEM((CHUNK,), jnp.int32),
        pltpu.VMEM((CHUNK, N_CLASSES), jnp.float32),
        pltpu.VMEM((REM,), jnp.int32),
        pltpu.VMEM((REM, N_CLASSES), jnp.float32),
    ]
    if nprev == 2:
        scratch.append(pltpu.VMEM((RPT, N_CLASSES), jnp.float32))

    @functools.partial(
        pl.kernel,
        out_type=jax.ShapeDtypeStruct((NC, N_NODES, N_CLASSES), jnp.float32),
        mesh=mesh,
        scratch_types=scratch,
    )
    def run(y_hbm, edge_hbm, out_hbm, y_sh, acc_sh, stage_a, srcbuf, dstbuf,
            idxd, rows, idxr, rowsr, *rest):
        cid = lax.axis_index("c")
        sid = lax.axis_index("s")
        wid = sid * NC + cid
        r0 = sid * RPT

        zvec = jnp.zeros((16,), jnp.float32)

        def zf(k, carry):
            stage_a[k // VPR, pl.ds((k % VPR) * 16, 16)] = zvec
            return carry

        lax.fori_loop(0, RPT * VPR, zf, 0)
        pltpu.sync_copy(stage_a, acc_sh.at[pl.ds(r0, RPT)])

        pltpu.sync_copy(y_hbm.at[0, pl.ds(r0, RPT)], stage_a)
        if nprev == 2:
            stage_b = rest[0]
            pltpu.sync_copy(y_hbm.at[1, pl.ds(r0, RPT)], stage_b)

            def addf(k, carry):
                i = k // VPR
                j = (k % VPR) * 16
                stage_a[i, pl.ds(j, 16)] = (
                    stage_a[i, pl.ds(j, 16)] + stage_b[i, pl.ds(j, 16)])
                return carry

            lax.fori_loop(0, RPT * VPR, addf, 0)
        pltpu.sync_copy(stage_a, y_sh.at[pl.ds(r0, RPT)])

        plsc.subcore_barrier()

        e0 = wid * E_W
        pltpu.sync_copy(edge_hbm.at[0, pl.ds(e0, E_W)], srcbuf)
        pltpu.sync_copy(edge_hbm.at[1, pl.ds(e0, E_W)], dstbuf)

        def chunk_step(c, carry):
            off = pl.multiple_of(c * CHUNK, 8)
            pltpu.sync_copy(y_sh.at[srcbuf.at[pl.ds(off, CHUNK)]], rows)
            pltpu.sync_copy(dstbuf.at[pl.ds(off, CHUNK)], idxd)
            pltpu.sync_copy(rows, acc_sh.at[idxd], add=True)
            return carry

        lax.fori_loop(0, NFULL, chunk_step, 0)

        offr = NFULL * CHUNK
        pltpu.sync_copy(y_sh.at[srcbuf.at[pl.ds(offr, REM)]], rowsr)
        pltpu.sync_copy(dstbuf.at[pl.ds(offr, REM)], idxr)
        pltpu.sync_copy(rowsr, acc_sh.at[idxr], add=True)

        plsc.subcore_barrier()

        pltpu.sync_copy(acc_sh.at[pl.ds(r0, RPT)],
                        out_hbm.at[cid, pl.ds(r0, RPT)])

    return run


_sc_round1 = _make_sc_round(1)
_sc_round2 = _make_sc_round(2)


def kernel(feat, edge_index, W, b):
    y0 = _tc_matmul(feat, W)
    p = _sc_round1(y0.reshape(1, N_NODES, N_CLASSES), edge_index)
    q = _sc_round2(p, edge_index)
    out = _tc_combine(q, jnp.broadcast_to(b, (1, N_CLASSES)))
    return out


# speedup vs baseline: 8.5244x; 8.5244x over previous
import functools

import jax
import jax.numpy as jnp
from jax import lax
from jax.experimental import pallas as pl
from jax.experimental.pallas import tpu as pltpu
from jax.experimental.pallas import tpu_sc as plsc

N_NODES = 10000
N_EDGES = 320000
D_FEAT = 128
N_CLASSES = 64

NC, NS = 2, 16
NW = NC * NS
E_W = N_EDGES // NW
CHUNK = 128
NFULL = E_W // CHUNK
REM = E_W - NFULL * CHUNK
N_PAD = 10240
RPT = N_PAD // NS
SCH = 160
NSCH = RPT // SCH
VPR = N_CLASSES // 16


def _mm_body(feat_ref, w_ref, o_ref):
    o_ref[...] = lax.dot_general(
        feat_ref[...], w_ref[...],
        (((1,), (1,)), ((), ())),
        preferred_element_type=jnp.float32,
    )


def _tc_matmul(featp, W):
    return pl.pallas_call(
        _mm_body,
        out_shape=jax.ShapeDtypeStruct((N_PAD, N_CLASSES), jnp.float32),
    )(featp, W)


def _comb_body(q_ref, b_ref, o_ref):
    o_ref[...] = (q_ref[0, :N_NODES, :] + q_ref[1, :N_NODES, :]
                  + b_ref[...])


def _tc_combine(q, b2):
    return pl.pallas_call(
        _comb_body,
        out_shape=jax.ShapeDtypeStruct((N_NODES, N_CLASSES), jnp.float32),
    )(q, b2)


def _make_sc_round(nprev: int):
    mesh = plsc.VectorSubcoreMesh(core_axis_name="c", subcore_axis_name="s")
    scratch = [
        pltpu.VMEM_SHARED((N_PAD, N_CLASSES), jnp.float32),
        pltpu.VMEM_SHARED((N_PAD, N_CLASSES), jnp.float32),
        pltpu.VMEM((SCH, N_CLASSES), jnp.float32),
        pltpu.VMEM((CHUNK,), jnp.int32),
        pltpu.VMEM((CHUNK,), jnp.int32),
        pltpu.VMEM((CHUNK, N_CLASSES), jnp.float32),
        pltpu.VMEM((REM,), jnp.int32),
        pltpu.VMEM((REM,), jnp.int32),
        pltpu.VMEM((REM, N_CLASSES), jnp.float32),
    ]
    if nprev == 2:
        scratch.append(pltpu.VMEM((SCH, N_CLASSES), jnp.float32))

    @functools.partial(
        pl.kernel,
        out_type=jax.ShapeDtypeStruct((NC, N_PAD, N_CLASSES), jnp.float32),
        mesh=mesh,
        scratch_types=scratch,
        compiler_params=pltpu.CompilerParams(use_tc_tiling_on_sc=False),
    )
    def run(y_hbm, src_hbm, dst_hbm, out_hbm, y_sh, acc_sh, stage_a, idxs,
            idxd, rows, idxsr, idxdr, rowsr, *rest):
        cid = lax.axis_index("c")
        sid = lax.axis_index("s")
        wid = sid * NC + cid
        r0 = sid * RPT

        zvec = jnp.zeros((16,), jnp.float32)

        def zf(k, carry):
            stage_a[k // VPR, pl.ds((k % VPR) * 16, 16)] = zvec
            return carry

        lax.fori_loop(0, SCH * VPR, zf, 0)

        def zcopy(t, carry):
            pltpu.sync_copy(stage_a, acc_sh.at[pl.ds(r0 + t * SCH, SCH)])
            return carry

        lax.fori_loop(0, NSCH, zcopy, 0)

        if nprev == 1:
            def st1(t, carry):
                rr = r0 + t * SCH
                pltpu.sync_copy(y_hbm.at[0, pl.ds(rr, SCH)], stage_a)
                pltpu.sync_copy(stage_a, y_sh.at[pl.ds(rr, SCH)])
                return carry

            lax.fori_loop(0, NSCH, st1, 0)
        else:
            stage_b = rest[0]

            def st2(t, carry):
                rr = r0 + t * SCH
                pltpu.sync_copy(y_hbm.at[0, pl.ds(rr, SCH)], stage_a)
                pltpu.sync_copy(y_hbm.at[1, pl.ds(rr, SCH)], stage_b)

                def addf(k, c2):
                    i = k // VPR
                    j = (k % VPR) * 16
                    stage_a[i, pl.ds(j, 16)] = (
                        stage_a[i, pl.ds(j, 16)] + stage_b[i, pl.ds(j, 16)])
                    return c2

                lax.fori_loop(0, SCH * VPR, addf, 0)
                pltpu.sync_copy(stage_a, y_sh.at[pl.ds(rr, SCH)])
                return carry

            lax.fori_loop(0, NSCH, st2, 0)

        plsc.subcore_barrier()

        def chunk_step(c, carry):
            off = pl.multiple_of(c * CHUNK, 8)
            pltpu.sync_copy(src_hbm.at[wid, 0, pl.ds(off, CHUNK)], idxs)
            pltpu.sync_copy(dst_hbm.at[wid, 0, pl.ds(off, CHUNK)], idxd)
            pltpu.sync_copy(y_sh.at[idxs], rows)
            pltpu.sync_copy(rows, acc_sh.at[idxd], add=True)
            return carry

        lax.fori_loop(0, NFULL, chunk_step, 0)

        offr = NFULL * CHUNK
        pltpu.sync_copy(src_hbm.at[wid, 0, pl.ds(offr, REM)], idxsr)
        pltpu.sync_copy(dst_hbm.at[wid, 0, pl.ds(offr, REM)], idxdr)
        pltpu.sync_copy(y_sh.at[idxsr], rowsr)
        pltpu.sync_copy(rowsr, acc_sh.at[idxdr], add=True)

        plsc.subcore_barrier()

        def wb(t, carry):
            rr = r0 + t * SCH
            pltpu.sync_copy(acc_sh.at[pl.ds(rr, SCH)], stage_a)
            pltpu.sync_copy(stage_a, out_hbm.at[cid, pl.ds(rr, SCH)])
            return carry

        lax.fori_loop(0, NSCH, wb, 0)

    return run


_sc_round1 = _make_sc_round(1)
_sc_round2 = _make_sc_round(2)


def kernel(feat, edge_index, W, b):
    featp = jnp.pad(feat, ((0, N_PAD - N_NODES), (0, 0)))
    src3 = edge_index[0].reshape(NW, 1, E_W)
    dst3 = edge_index[1].reshape(NW, 1, E_W)
    y0 = _tc_matmul(featp, W)
    p = _sc_round1(y0.reshape(1, N_PAD, N_CLASSES), src3, dst3)
    q = _sc_round2(p, src3, dst3)
    out = _tc_combine(q, jnp.broadcast_to(b, (1, N_CLASSES)))
    return out


# speedup vs baseline: 14.1303x; 1.6576x over previous
import functools

import jax
import jax.numpy as jnp
from jax import lax
from jax.experimental import pallas as pl
from jax.experimental.pallas import tpu as pltpu
from jax.experimental.pallas import tpu_sc as plsc

N_NODES = 10000
N_EDGES = 320000
D_FEAT = 128
N_CLASSES = 64

NC, NS = 2, 16
NW = NC * NS
CHUNK = 128
NCH = 80
E_PK = NW * NCH * CHUNK
PADE = E_PK - N_EDGES
N_PAD = 10240
RPT = N_PAD // NS
DEPTH = 4
NG = NCH // DEPTH


def _mm_body(feat_ref, w_ref, o_ref):
    o_ref[...] = lax.dot_general(
        feat_ref[...], w_ref[...],
        (((1,), (1,)), ((), ())),
        preferred_element_type=jnp.float32,
    )


def _tc_matmul(featp, W):
    return pl.pallas_call(
        _mm_body,
        out_shape=jax.ShapeDtypeStruct((N_PAD, N_CLASSES), jnp.float32),
    )(featp, W)


def _mid_body(p_ref, o_ref):
    o_ref[...] = p_ref[0] + p_ref[1]


def _tc_mid(p):
    return pl.pallas_call(
        _mid_body,
        out_shape=jax.ShapeDtypeStruct((N_PAD, N_CLASSES), jnp.float32),
    )(p)


def _comb_body(q_ref, b_ref, o_ref):
    o_ref[...] = (q_ref[0, :N_NODES, :] + q_ref[1, :N_NODES, :]
                  + b_ref[...])


def _tc_combine(q, b2):
    return pl.pallas_call(
        _comb_body,
        out_shape=jax.ShapeDtypeStruct((N_NODES, N_CLASSES), jnp.float32),
    )(q, b2)


def _make_sc_hop():
    mesh = plsc.VectorSubcoreMesh(core_axis_name="c", subcore_axis_name="s")
    scratch = (
        [pltpu.VMEM_SHARED((N_PAD, N_CLASSES), jnp.float32)] * 2
        + [pltpu.VMEM((2, CHUNK), jnp.int32)] * DEPTH
        + [pltpu.VMEM((CHUNK, N_CLASSES), jnp.float32)] * DEPTH
        + [pltpu.SemaphoreType.DMA] * (3 + 3 * DEPTH)
    )

    @functools.partial(
        pl.kernel,
        out_type=jax.ShapeDtypeStruct((NC, N_PAD, N_CLASSES), jnp.float32),
        mesh=mesh,
        scratch_types=scratch,
        compiler_params=pltpu.CompilerParams(use_tc_tiling_on_sc=False),
    )
    def run(y_hbm, z_hbm, epk_hbm, out_hbm, y_sh, acc_sh, *rest):
        idx2 = rest[:DEPTH]
        rows = rest[DEPTH:2 * DEPTH]
        sem_y, sem_z, sem_w = rest[2 * DEPTH:2 * DEPTH + 3]
        sem_i = rest[2 * DEPTH + 3:2 * DEPTH + 3 + DEPTH]
        sem_g = rest[2 * DEPTH + 3 + DEPTH:2 * DEPTH + 3 + 2 * DEPTH]
        sem_s = rest[2 * DEPTH + 3 + 2 * DEPTH:]

        cid = lax.axis_index("c")
        sid = lax.axis_index("s")
        wid = sid * NC + cid
        r0 = sid * RPT

        dy = pltpu.async_copy(y_hbm.at[pl.ds(r0, RPT)],
                              y_sh.at[pl.ds(r0, RPT)], sem_y)
        dz = pltpu.async_copy(z_hbm.at[pl.ds(r0, RPT)],
                              acc_sh.at[pl.ds(r0, RPT)], sem_z)
        for j in range(DEPTH):
            pltpu.async_copy(epk_hbm.at[wid, j], idx2[j], sem_i[j])
        dy.wait()
        dz.wait()
        plsc.subcore_barrier()

        def grp(g, carry):
            gd = []
            for j in range(DEPTH):
                pltpu.make_async_copy(epk_hbm.at[wid, 0], idx2[j],
                                      sem_i[j]).wait()
                gd.append(pltpu.async_copy(y_sh.at[idx2[j].at[0]], rows[j],
                                           sem_g[j]))
            sd = []
            for j in range(DEPTH):
                gd[j].wait()
                sd.append(pltpu.async_copy(rows[j], acc_sh.at[idx2[j].at[1]],
                                           sem_s[j], add=True))
            for j in range(DEPTH):
                sd[j].wait()

            @pl.when(g < NG - 1)
            def _():
                for j in range(DEPTH):
                    pltpu.async_copy(epk_hbm.at[wid, (g + 1) * DEPTH + j],
                                     idx2[j], sem_i[j])

            return carry

        lax.fori_loop(0, NG, grp, 0)
        plsc.subcore_barrier()

        pltpu.async_copy(acc_sh.at[pl.ds(r0, RPT)],
                         out_hbm.at[cid, pl.ds(r0, RPT)], sem_w).wait()

    return run


_sc_hop = _make_sc_hop()


def kernel(feat, edge_index, W, b):
    featp = jnp.pad(feat, ((0, N_PAD - N_NODES), (0, 0)))
    fill = jnp.arange(PADE, dtype=jnp.int32) % (N_PAD - N_NODES)
    srcp = jnp.concatenate([edge_index[0], fill])
    dstp = jnp.concatenate([edge_index[1], N_NODES + fill])
    epk = jnp.stack([srcp.reshape(NW, NCH, CHUNK),
                     dstp.reshape(NW, NCH, CHUNK)], axis=2)
    z = jnp.zeros((N_PAD, N_CLASSES), jnp.float32)

    y0 = _tc_matmul(featp, W)
    p = _sc_hop(y0, z, epk)
    y1 = _tc_mid(p)
    q = _sc_hop(y1, z, epk)
    out = _tc_combine(q, jnp.broadcast_to(b, (1, N_CLASSES)))
    return out
